# baseline (speedup 1.0000x reference)
import functools

import jax
import jax.numpy as jnp
from jax import lax
from jax.experimental import pallas as pl
from jax.experimental.pallas import tpu as pltpu
from jax.experimental.pallas import tpu_sc as plsc

N = 50000
E = 800000
B = 256
F = 64
HID = 64
N_GCN = 4
NC = 2
NS = 16
HALF = N // NC
SLICE = 1568
ACC_ROWS = NS * SLICE
TRASH = HALF
CH = 128
NCHUNKS = E // CH
HISTP = 25024
RB = 2000

_mesh = plsc.VectorSubcoreMesh(core_axis_name="c", subcore_axis_name="s")


CHA = 192
PADM = 4 * CHA
T0 = NCHUNKS // NS
NTAIL = NCHUNKS - T0 * NS
FLUSH = 1024
CAPB = 2048
PCAP = 51200


def _tile_ids():
    c = lax.axis_index("c")
    s = lax.axis_index("s")
    return c, s


def _zero_acc_slice(zeros_hbm, acc, s):
    pltpu.sync_copy(zeros_hbm, acc.at[pl.ds(s * SLICE, SLICE)])


def _writeback(acc, out_hbm, c, s):
    lo = jnp.minimum(s * SLICE, HALF - SLICE)
    pltpu.sync_copy(acc.at[pl.ds(lo, SLICE)],
                    out_hbm.at[pl.ds(c * HALF + lo, SLICE)])


def _compute_ldst(dstv, ldst, k, base):
    for q in range(CH // 16):
        d = dstv[k, pl.ds(q * 16, 16)]
        l = d - base
        ok = (l >= 0) & (l < HALF)
        ldst[k, pl.ds(q * 16, 16)] = jnp.where(ok, l, TRASH)


def _sc_prep_body(src_hbm, dst_hbm, deg_hbm, psrc_hbm,
                  pldst_hbm, cnt_hbm, srcv, dstv, ldst, obs, obl,
                  cntv, hist, rbuf, wbuf, acc2, semL0, semL1):
    c, s = _tile_ids()
    base = c * HALF
    semL = [semL0, semL1]
    fones = jnp.ones((16,), jnp.float32)

    def zbody(i, carry0):
        off = pl.multiple_of(i * 16, 16)
        hist[pl.ds(off, 16)] = jnp.zeros((16,), jnp.float32)
        return carry0

    lax.fori_loop(0, HISTP // 16, zbody, 0)

    def load(t, k):
        off = (t * NS + s) * CH
        pltpu.async_copy(src_hbm.at[pl.ds(off, CH)], srcv.at[k], semL[k])
        pltpu.async_copy(dst_hbm.at[pl.ds(off, CH)], dstv.at[k], semL[k])

    def wait_load(k):
        pltpu.make_async_copy(src_hbm.at[pl.ds(0, CH)], srcv.at[k],
                              semL[k]).wait()
        pltpu.make_async_copy(dst_hbm.at[pl.ds(0, CH)], dstv.at[k],
                              semL[k]).wait()

    def compact(k, p, wp):
        for q in range(CH // 16):
            sv = srcv[k, pl.ds(q * 16, 16)]
            lv = ldst[k, pl.ds(q * 16, 16)]
            m = lv < TRASH
            plsc.addupdate_scatter(hist, [lv], fones)
            cs = plsc.cumsum(m.astype(jnp.int32))
            pos = p + cs - 1
            plsc.store_scatter(obs, [pos], sv, mask=m)
            plsc.store_scatter(obl, [pos], lv, mask=m)
            p = p + cs[15]
        do_flush = p >= FLUSH

        @pl.when(do_flush)
        def _():
            wpa = pl.multiple_of(wp, FLUSH)
            pltpu.sync_copy(obs.at[pl.ds(0, FLUSH)],
                            psrc_hbm.at[c, s, pl.ds(wpa, FLUSH)])
            pltpu.sync_copy(obl.at[pl.ds(0, FLUSH)],
                            pldst_hbm.at[c, s, pl.ds(wpa, FLUSH)])
            for gg in range((CAPB - FLUSH) // 16):
                obs[pl.ds(gg * 16, 16)] = obs[pl.ds(FLUSH + gg * 16, 16)]
                obl[pl.ds(gg * 16, 16)] = obl[pl.ds(FLUSH + gg * 16, 16)]

        p = jnp.where(do_flush, p - FLUSH, p)
        wp = jnp.where(do_flush, wp + FLUSH, wp)
        return p, wp

    load(0, 0)

    def body(o, carry):
        p, wp = carry
        for r in range(2):
            t = o * 2 + r

            @pl.when(t < T0 - 1)
            def _():
                load(t + 1, 1 - r)

            wait_load(r)
            _compute_ldst(dstv, ldst, r, base)
            p, wp = compact(r, p, wp)
        return p, wp

    p, wp = lax.fori_loop(0, T0 // 2, body, (jnp.int32(0), jnp.int32(0)))

    @pl.when(s < NTAIL)
    def _():
        off = (T0 * NS + s) * CH
        pltpu.sync_copy(src_hbm.at[pl.ds(off, CH)], srcv.at[0])
        pltpu.sync_copy(dst_hbm.at[pl.ds(off, CH)], dstv.at[0])

    @pl.when(s >= NTAIL)
    def _():
        for q in range(CH // 16):
            dstv[0, pl.ds(q * 16, 16)] = jnp.full((16,), N, jnp.int32)

    _compute_ldst(dstv, ldst, 0, base)

    p, wp = compact(0, p, wp)

    total = wp + p
    total_pad = ((jnp.maximum(total, 1) + (PADM - 1)) // (PADM)
                 ) * (PADM)
    ngrp = (total_pad - total + 15) // 16

    def padbody(i, carry2):
        pos = p + i * 16 + lax.broadcasted_iota(jnp.int32, (16,), 0)
        plsc.store_scatter(obs, [pos], jnp.zeros((16,), jnp.int32))
        plsc.store_scatter(obl, [pos], jnp.full((16,), TRASH, jnp.int32))
        return carry2

    lax.fori_loop(0, ngrp, padbody, 0)
    wpa = pl.multiple_of(wp, FLUSH)
    pltpu.sync_copy(obs, psrc_hbm.at[c, s, pl.ds(wpa, CAPB)])
    pltpu.sync_copy(obl, pldst_hbm.at[c, s, pl.ds(wpa, CAPB)])
    cntv[...] = jnp.full((16,), total_pad, jnp.int32)
    pltpu.sync_copy(cntv, cnt_hbm.at[c, s])

    pltpu.sync_copy(hist, acc2.at[s])
    plsc.subcore_barrier()
    lo = pl.multiple_of(jnp.minimum(s * SLICE, HALF - SLICE), 8)
    for k in range(NS):
        pltpu.sync_copy(acc2.at[k, pl.ds(lo, SLICE)], rbuf.at[k])

    def redbody(j, carry3):
        off = pl.multiple_of(j * 16, 16)
        tot = rbuf[0, pl.ds(off, 16)]
        for k in range(1, NS):
            tot = tot + rbuf[k, pl.ds(off, 16)]
        wbuf[pl.ds(off, 16)] = tot
        return carry3

    lax.fori_loop(0, SLICE // 16, redbody, 0)
    pltpu.sync_copy(wbuf, deg_hbm.at[pl.ds(c * HALF + lo, SLICE)])


def _sc_agg_body(g_hbm, psrc_hbm, pldst_hbm, cnt_hbm, zeros_hbm, agg_hbm,
                 srcv, ldst, staged, cntv, acc, semL0, semL1, semL2, semL3,
                 semG0, semG1, semS0, semS1):
    c, s = _tile_ids()
    semL = [semL0, semL1, semL2, semL3]
    semG = [semG0, semG1]
    semS = [semS0, semS1]
    pltpu.sync_copy(cnt_hbm.at[c, s], cntv)
    cnt = cntv[...][0]
    _zero_acc_slice(zeros_hbm, acc, s)
    plsc.subcore_barrier()
    nch = cnt // CHA

    def load(t, il):
        off = t * CHA
        pltpu.async_copy(psrc_hbm.at[c, s, pl.ds(off, CHA)], srcv.at[il],
                         semL[il])
        pltpu.async_copy(pldst_hbm.at[c, s, pl.ds(off, CHA)], ldst.at[il],
                         semL[il])

    def wait_load(il):
        pltpu.make_async_copy(psrc_hbm.at[c, s, pl.ds(0, CHA)], srcv.at[il],
                              semL[il]).wait()
        pltpu.make_async_copy(pldst_hbm.at[c, s, pl.ds(0, CHA)], ldst.at[il],
                              semL[il]).wait()

    def gath(il, st):
        pltpu.async_copy(g_hbm.at[srcv.at[il]], staged.at[st], semG[st])

    def wait_gath(il, st):
        pltpu.make_async_copy(g_hbm.at[srcv.at[il]], staged.at[st],
                              semG[st]).wait()

    def scat(il, st):
        pltpu.async_copy(staged.at[st], acc.at[ldst.at[il]], semS[st],
                         add=True)

    def wait_scat(il, st):
        pltpu.make_async_copy(staged.at[st], acc.at[ldst.at[il]],
                              semS[st]).wait()

    load(0, 0)
    load(1, 1)

    def body(o, carry):
        for rr in range(4):
            t = o * 4 + rr
            st = rr % 2

            @pl.when(t >= 2)
            def _():
                wait_scat((rr + 2) % 4, st)

            @pl.when(t + 2 < nch)
            def _():
                load(t + 2, (rr + 2) % 4)

            wait_load(rr)
            gath(rr, st)

            @pl.when(t >= 1)
            def _():
                wait_gath((rr + 3) % 4, (rr + 1) % 2)
                scat((rr + 3) % 4, (rr + 1) % 2)

        return carry

    lax.fori_loop(0, nch // 4, body, 0)
    wait_gath(3, 1)
    scat(3, 1)
    wait_scat(2, 0)
    wait_scat(3, 1)

    plsc.subcore_barrier()
    _writeback(acc, agg_hbm, c, s)


_sc_params = pltpu.CompilerParams(use_tc_tiling_on_sc=False,
                                  needs_layout_passes=False)

_sc_prep = pl.kernel(
    _sc_prep_body,
    out_type=(
        jax.ShapeDtypeStruct((N,), jnp.float32),
        jax.ShapeDtypeStruct((NC, NS, PCAP), jnp.int32),
        jax.ShapeDtypeStruct((NC, NS, PCAP), jnp.int32),
        jax.ShapeDtypeStruct((NC, NS, 16), jnp.int32),
    ),
    mesh=_mesh,
    compiler_params=_sc_params,
    scratch_types=[
        pltpu.VMEM((2, CH), jnp.int32),
        pltpu.VMEM((2, CH), jnp.int32),
        pltpu.VMEM((2, CH), jnp.int32),
        pltpu.VMEM((CAPB,), jnp.int32),
        pltpu.VMEM((CAPB,), jnp.int32),
        pltpu.VMEM((16,), jnp.int32),
        pltpu.VMEM((HISTP,), jnp.float32),
        pltpu.VMEM((NS, SLICE), jnp.float32),
        pltpu.VMEM((SLICE,), jnp.float32),
        pltpu.VMEM_SHARED((NS, HISTP), jnp.float32),
    ] + [pltpu.SemaphoreType.DMA] * 2,
)

_sc_agg = pl.kernel(
    _sc_agg_body,
    out_type=jax.ShapeDtypeStruct((N, F), jnp.float32),
    mesh=_mesh,
    compiler_params=_sc_params,
    scratch_types=[
        pltpu.VMEM((4, CHA), jnp.int32),
        pltpu.VMEM((4, CHA), jnp.int32),
        pltpu.VMEM((2, CHA, F), jnp.float32),
        pltpu.VMEM((16,), jnp.int32),
        pltpu.VMEM_SHARED((ACC_ROWS, F), jnp.float32),
    ] + [pltpu.SemaphoreType.DMA] * 8,
)


def _tc_init_body(solv_ref, wemb_ref, bemb_ref, deg_ref, h_ref, g_ref,
                  dinv_ref):
    dinv = lax.rsqrt(jnp.maximum(deg_ref[...], 1.0))
    h = jnp.dot(solv_ref[...], wemb_ref[...]) + bemb_ref[...]
    h_ref[...] = h
    g_ref[...] = h * dinv
    dinv_ref[...] = dinv


_tc_init = pl.pallas_call(
    _tc_init_body,
    grid=(N // RB,),
    in_specs=[
        pl.BlockSpec((RB, 128), lambda i: (i, 0)),
        pl.BlockSpec((128, F), lambda i: (0, 0)),
        pl.BlockSpec((1, F), lambda i: (0, 0)),
        pl.BlockSpec((RB, 1), lambda i: (i, 0)),
    ],
    out_specs=[
        pl.BlockSpec((RB, F), lambda i: (i, 0)),
        pl.BlockSpec((RB, F), lambda i: (i, 0)),
        pl.BlockSpec((RB, 1), lambda i: (i, 0)),
    ],
    out_shape=[
        jax.ShapeDtypeStruct((N, F), jnp.float32),
        jax.ShapeDtypeStruct((N, F), jnp.float32),
        jax.ShapeDtypeStruct((N, 1), jnp.float32),
    ],
)


def _tc_layer_body(agg_ref, h_ref, dinv_ref, w_ref, b_ref, hn_ref, gn_ref):
    dinv = dinv_ref[...]
    a = agg_ref[...] * dinv
    z = jnp.dot(a, w_ref[...]) + b_ref[...]
    hn = h_ref[...] + jnp.maximum(z, 0.0)
    hn_ref[...] = hn
    gn_ref[...] = hn * dinv


_tc_layer = pl.pallas_call(
    _tc_layer_body,
    grid=(N // RB,),
    in_specs=[
        pl.BlockSpec((RB, F), lambda i: (i, 0)),
        pl.BlockSpec((RB, F), lambda i: (i, 0)),
        pl.BlockSpec((RB, 1), lambda i: (i, 0)),
        pl.BlockSpec((F, F), lambda i: (0, 0)),
        pl.BlockSpec((1, F), lambda i: (0, 0)),
    ],
    out_specs=[
        pl.BlockSpec((RB, F), lambda i: (i, 0)),
        pl.BlockSpec((RB, F), lambda i: (i, 0)),
    ],
    out_shape=[
        jax.ShapeDtypeStruct((N, F), jnp.float32),
        jax.ShapeDtypeStruct((N, F), jnp.float32),
    ],
)


def _tc_final_body(ids_ref, h_ref, hidden_ref, l2w0_ref, l2b0_ref, l2w1_ref,
                   l2b1_ref, w1_ref, b1_ref, w2_ref, b2_ref, out_ref,
                   pooled_scr):
    i = pl.program_id(0)

    @pl.when(i == 0)
    def _():
        pooled_scr[...] = jnp.zeros_like(pooled_scr)

    onehot = (ids_ref[...] == lax.broadcasted_iota(jnp.int32, (1, B), 1)
              ).astype(jnp.float32)
    pooled_scr[...] += lax.dot_general(onehot, h_ref[...],
                                       (((0,), (0,)), ((), ())))

    @pl.when(i == pl.num_programs(0) - 1)
    def _():
        p = pooled_scr[...]
        p = jnp.maximum(jnp.dot(p, l2w0_ref[...]) + l2b0_ref[...], 0.0)
        p = jnp.maximum(jnp.dot(p, l2w1_ref[...]) + l2b1_ref[...], 0.0)
        hc = jnp.concatenate([hidden_ref[...], p], axis=1)
        hc = jnp.maximum(jnp.dot(hc, w1_ref[...]) + b1_ref[...], 0.0)
        out_ref[...] = jnp.dot(hc, w2_ref[...]) + b2_ref[...]


_tc_final = pl.pallas_call(
    _tc_final_body,
    grid=(N // RB,),
    in_specs=[
        pl.BlockSpec((RB, 1), lambda i: (i, 0)),
        pl.BlockSpec((RB, F), lambda i: (i, 0)),
        pl.BlockSpec((B, HID), lambda i: (0, 0)),
        pl.BlockSpec((F, F), lambda i: (0, 0)),
        pl.BlockSpec((1, F), lambda i: (0, 0)),
        pl.BlockSpec((F, F), lambda i: (0, 0)),
        pl.BlockSpec((1, F), lambda i: (0, 0)),
        pl.BlockSpec((HID + F, HID + F), lambda i: (0, 0)),
        pl.BlockSpec((1, HID + F), lambda i: (0, 0)),
        pl.BlockSpec((HID + F, F), lambda i: (0, 0)),
        pl.BlockSpec((1, F), lambda i: (0, 0)),
    ],
    out_specs=pl.BlockSpec((B, F), lambda i: (0, 0)),
    out_shape=jax.ShapeDtypeStruct((B, F), jnp.float32),
    scratch_shapes=[pltpu.VMEM((B, F), jnp.float32)],
)


def kernel(hidden_feats, solv_node_feats, edge_index, node_graph_ids, W_emb,
           b_emb, gcn_W, gcn_b, lin2_W, lin2_b, lin3_W1, lin3_b1, lin3_W2,
           lin3_b2):
    edge = edge_index.astype(jnp.int32)
    src = edge[0]
    dst = edge[1]
    ids = node_graph_ids.astype(jnp.int32).reshape(N, 1)
    zerosF = jnp.zeros((SLICE, F), jnp.float32)

    deg, psrc, pldst, cnt = _sc_prep(src, dst)
    h, g, dinv = _tc_init(solv_node_feats, W_emb, b_emb.reshape(1, F),
                          deg.reshape(N, 1))
    for i in range(N_GCN):
        agg = _sc_agg(g, psrc, pldst, cnt, zerosF)
        h, g = _tc_layer(agg, h, dinv, gcn_W[i], gcn_b[i].reshape(1, F))
    out = _tc_final(ids, h, hidden_feats, lin2_W[0], lin2_b[0].reshape(1, F),
                    lin2_W[1], lin2_b[1].reshape(1, F), lin3_W1,
                    lin3_b1.reshape(1, HID + F), lin3_W2,
                    lin3_b2.reshape(1, F))
    return out


# speedup vs baseline: 1.1523x; 1.1523x over previous
import functools

import jax
import jax.numpy as jnp
from jax import lax
from jax.experimental import pallas as pl
from jax.experimental.pallas import tpu as pltpu
from jax.experimental.pallas import tpu_sc as plsc

N = 50000
E = 800000
B = 256
F = 64
HID = 64
N_GCN = 4
NC = 2
NS = 16
HALF = N // NC
SLICE = 1568
ACC_ROWS = NS * SLICE
TRASH = HALF
CH = 128
NCHUNKS = E // CH
HISTP = 25024
RB = 2000

_mesh = plsc.VectorSubcoreMesh(core_axis_name="c", subcore_axis_name="s")


RD = 4
CHA = 96
T0 = NCHUNKS // NS
NTAIL = NCHUNKS - T0 * NS
FLUSH = 1024
CAPB = 1536
PCAP = 51200


def _tile_ids():
    c = lax.axis_index("c")
    s = lax.axis_index("s")
    return c, s


def _zero_acc_slice(zeros_hbm, acc, s):
    pltpu.sync_copy(zeros_hbm, acc.at[pl.ds(s * SLICE, SLICE)])


def _writeback(acc, out_hbm, c, s):
    lo = jnp.minimum(s * SLICE, HALF - SLICE)
    pltpu.sync_copy(acc.at[pl.ds(lo, SLICE)],
                    out_hbm.at[pl.ds(c * HALF + lo, SLICE)])


def _compute_ldst(dstv, ldst, k, base):
    for q in range(CH // 16):
        d = dstv[k, pl.ds(q * 16, 16)]
        l = d - base
        ok = (l >= 0) & (l < HALF)
        ldst[k, pl.ds(q * 16, 16)] = jnp.where(ok, l, TRASH)


def _sc_prep_body(src_hbm, dst_hbm, deg_hbm, psrc_hbm,
                  pldst_hbm, cnt_hbm, srcv, dstv, ldst, obs, obl,
                  cntv, hist, rbuf, wbuf, acc2, semL0, semL1):
    c, s = _tile_ids()
    base = c * HALF
    semL = [semL0, semL1]
    fones = jnp.ones((16,), jnp.float32)

    def zbody(i, carry0):
        off = pl.multiple_of(i * 16, 16)
        hist[pl.ds(off, 16)] = jnp.zeros((16,), jnp.float32)
        return carry0

    lax.fori_loop(0, HISTP // 16, zbody, 0)

    def load(t, k):
        off = (t * NS + s) * CH
        pltpu.async_copy(src_hbm.at[pl.ds(off, CH)], srcv.at[k], semL[k])
        pltpu.async_copy(dst_hbm.at[pl.ds(off, CH)], dstv.at[k], semL[k])

    def wait_load(k):
        pltpu.make_async_copy(src_hbm.at[pl.ds(0, CH)], srcv.at[k],
                              semL[k]).wait()
        pltpu.make_async_copy(dst_hbm.at[pl.ds(0, CH)], dstv.at[k],
                              semL[k]).wait()

    def compact(k, p, wp):
        for q in range(CH // 16):
            sv = srcv[k, pl.ds(q * 16, 16)]
            lv = ldst[k, pl.ds(q * 16, 16)]
            m = lv < TRASH
            plsc.addupdate_scatter(hist, [lv], fones)
            cs = plsc.cumsum(m.astype(jnp.int32))
            pos = p + cs - 1
            plsc.store_scatter(obs, [pos], sv, mask=m)
            plsc.store_scatter(obl, [pos], lv, mask=m)
            p = p + cs[15]
        do_flush = p >= FLUSH

        @pl.when(do_flush)
        def _():
            wpa = pl.multiple_of(wp, FLUSH)
            pltpu.sync_copy(obs.at[pl.ds(0, FLUSH)],
                            psrc_hbm.at[c, s, pl.ds(wpa, FLUSH)])
            pltpu.sync_copy(obl.at[pl.ds(0, FLUSH)],
                            pldst_hbm.at[c, s, pl.ds(wpa, FLUSH)])
            for gg in range((CAPB - FLUSH) // 16):
                obs[pl.ds(gg * 16, 16)] = obs[pl.ds(FLUSH + gg * 16, 16)]
                obl[pl.ds(gg * 16, 16)] = obl[pl.ds(FLUSH + gg * 16, 16)]

        p = jnp.where(do_flush, p - FLUSH, p)
        wp = jnp.where(do_flush, wp + FLUSH, wp)
        return p, wp

    load(0, 0)

    def body(o, carry):
        p, wp = carry
        for r in range(2):
            t = o * 2 + r

            @pl.when(t < T0 - 1)
            def _():
                load(t + 1, 1 - r)

            wait_load(r)
            _compute_ldst(dstv, ldst, r, base)
            p, wp = compact(r, p, wp)
        return p, wp

    p, wp = lax.fori_loop(0, T0 // 2, body, (jnp.int32(0), jnp.int32(0)))

    @pl.when(s < NTAIL)
    def _():
        off = (T0 * NS + s) * CH
        pltpu.sync_copy(src_hbm.at[pl.ds(off, CH)], srcv.at[0])
        pltpu.sync_copy(dst_hbm.at[pl.ds(off, CH)], dstv.at[0])

    @pl.when(s >= NTAIL)
    def _():
        for q in range(CH // 16):
            dstv[0, pl.ds(q * 16, 16)] = jnp.full((16,), N, jnp.int32)

    _compute_ldst(dstv, ldst, 0, base)

    p, wp = compact(0, p, wp)

    total = wp + p
    total_pad = ((jnp.maximum(total, 1) + (RD * CHA - 1)) // (RD * CHA)
                 ) * (RD * CHA)
    ngrp = (total_pad - total + 15) // 16

    def padbody(i, carry2):
        pos = p + i * 16 + lax.broadcasted_iota(jnp.int32, (16,), 0)
        plsc.store_scatter(obs, [pos], jnp.zeros((16,), jnp.int32))
        plsc.store_scatter(obl, [pos], jnp.full((16,), TRASH, jnp.int32))
        return carry2

    lax.fori_loop(0, ngrp, padbody, 0)
    wpa = pl.multiple_of(wp, FLUSH)
    pltpu.sync_copy(obs, psrc_hbm.at[c, s, pl.ds(wpa, CAPB)])
    pltpu.sync_copy(obl, pldst_hbm.at[c, s, pl.ds(wpa, CAPB)])
    cntv[...] = jnp.full((16,), total_pad, jnp.int32)
    pltpu.sync_copy(cntv, cnt_hbm.at[c, s])

    pltpu.sync_copy(hist, acc2.at[s])
    plsc.subcore_barrier()
    lo = pl.multiple_of(jnp.minimum(s * SLICE, HALF - SLICE), 8)
    for k in range(NS):
        pltpu.sync_copy(acc2.at[k, pl.ds(lo, SLICE)], rbuf.at[k])

    def redbody(j, carry3):
        off = pl.multiple_of(j * 16, 16)
        tot = rbuf[0, pl.ds(off, 16)]
        for k in range(1, NS):
            tot = tot + rbuf[k, pl.ds(off, 16)]
        wbuf[pl.ds(off, 16)] = tot
        return carry3

    lax.fori_loop(0, SLICE // 16, redbody, 0)
    pltpu.sync_copy(wbuf, deg_hbm.at[pl.ds(c * HALF + lo, SLICE)])


def _sc_agg_body(g_hbm, psrc_hbm, pldst_hbm, cnt_hbm, zeros_hbm, agg_hbm,
                 srcv, ldst, staged, cntv, acc, semL0, semL1, semL2, semL3,
                 semG0, semG1, semG2, semG3, semS0, semS1, semS2, semS3):
    c, s = _tile_ids()
    semL = [semL0, semL1, semL2, semL3]
    semG = [semG0, semG1, semG2, semG3]
    semS = [semS0, semS1, semS2, semS3]
    pltpu.sync_copy(cnt_hbm.at[c, s], cntv)
    cnt = cntv[...][0]
    _zero_acc_slice(zeros_hbm, acc, s)
    plsc.subcore_barrier()

    def load(t, k):
        off = t * CHA
        pltpu.async_copy(psrc_hbm.at[c, s, pl.ds(off, CHA)], srcv.at[k],
                         semL[k])
        pltpu.async_copy(pldst_hbm.at[c, s, pl.ds(off, CHA)], ldst.at[k],
                         semL[k])

    def wait_load(k):
        pltpu.make_async_copy(psrc_hbm.at[c, s, pl.ds(0, CHA)], srcv.at[k],
                              semL[k]).wait()
        pltpu.make_async_copy(pldst_hbm.at[c, s, pl.ds(0, CHA)], ldst.at[k],
                              semL[k]).wait()

    def gath(k):
        pltpu.async_copy(g_hbm.at[srcv.at[k]], staged.at[k], semG[k])

    def wait_gath(k):
        pltpu.make_async_copy(g_hbm.at[srcv.at[k]], staged.at[k],
                              semG[k]).wait()

    def scat(k):
        pltpu.async_copy(staged.at[k], acc.at[ldst.at[k]], semS[k], add=True)

    def wait_scat(k):
        pltpu.make_async_copy(staged.at[k], acc.at[ldst.at[k]],
                              semS[k]).wait()

    def body(o, carry):
        for r in range(RD):
            t = o * RD + r

            @pl.when(t >= RD)
            def _():
                wait_scat(r)

            load(t, r)

            @pl.when(t >= 1)
            def _():
                k1 = (r + RD - 1) % RD
                wait_load(k1)
                gath(k1)

            @pl.when(t >= 2)
            def _():
                k2 = (r + RD - 2) % RD
                wait_gath(k2)
                scat(k2)

        return carry

    lax.fori_loop(0, cnt // (RD * CHA), body, 0)
    wait_load(RD - 1)
    gath(RD - 1)
    wait_gath(RD - 2)
    scat(RD - 2)
    wait_gath(RD - 1)
    scat(RD - 1)
    for r in range(RD):
        wait_scat(r)

    plsc.subcore_barrier()
    _writeback(acc, agg_hbm, c, s)


_sc_params = pltpu.CompilerParams(use_tc_tiling_on_sc=False,
                                  needs_layout_passes=False)

_sc_prep = pl.kernel(
    _sc_prep_body,
    out_type=(
        jax.ShapeDtypeStruct((N,), jnp.float32),
        jax.ShapeDtypeStruct((NC, NS, PCAP), jnp.int32),
        jax.ShapeDtypeStruct((NC, NS, PCAP), jnp.int32),
        jax.ShapeDtypeStruct((NC, NS, 16), jnp.int32),
    ),
    mesh=_mesh,
    compiler_params=_sc_params,
    scratch_types=[
        pltpu.VMEM((2, CH), jnp.int32),
        pltpu.VMEM((2, CH), jnp.int32),
        pltpu.VMEM((2, CH), jnp.int32),
        pltpu.VMEM((CAPB,), jnp.int32),
        pltpu.VMEM((CAPB,), jnp.int32),
        pltpu.VMEM((16,), jnp.int32),
        pltpu.VMEM((HISTP,), jnp.float32),
        pltpu.VMEM((NS, SLICE), jnp.float32),
        pltpu.VMEM((SLICE,), jnp.float32),
        pltpu.VMEM_SHARED((NS, HISTP), jnp.float32),
    ] + [pltpu.SemaphoreType.DMA] * 2,
)

_sc_agg = pl.kernel(
    _sc_agg_body,
    out_type=jax.ShapeDtypeStruct((N, F), jnp.float32),
    mesh=_mesh,
    compiler_params=_sc_params,
    scratch_types=[
        pltpu.VMEM((RD, CHA), jnp.int32),
        pltpu.VMEM((RD, CHA), jnp.int32),
        pltpu.VMEM((RD, CHA, F), jnp.float32),
        pltpu.VMEM((16,), jnp.int32),
        pltpu.VMEM_SHARED((ACC_ROWS, F), jnp.float32),
    ] + [pltpu.SemaphoreType.DMA] * 12,
)


def _tc_init_body(solv_ref, wemb_ref, bemb_ref, deg_ref, h_ref, g_ref,
                  dinv_ref):
    dinv = lax.rsqrt(jnp.maximum(deg_ref[...], 1.0))
    h = jnp.dot(solv_ref[...], wemb_ref[...]) + bemb_ref[...]
    h_ref[...] = h
    g_ref[...] = h * dinv
    dinv_ref[...] = dinv


_tc_init = pl.pallas_call(
    _tc_init_body,
    grid=(N // RB,),
    in_specs=[
        pl.BlockSpec((RB, 128), lambda i: (i, 0)),
        pl.BlockSpec((128, F), lambda i: (0, 0)),
        pl.BlockSpec((1, F), lambda i: (0, 0)),
        pl.BlockSpec((RB, 1), lambda i: (i, 0)),
    ],
    out_specs=[
        pl.BlockSpec((RB, F), lambda i: (i, 0)),
        pl.BlockSpec((RB, F), lambda i: (i, 0)),
        pl.BlockSpec((RB, 1), lambda i: (i, 0)),
    ],
    out_shape=[
        jax.ShapeDtypeStruct((N, F), jnp.float32),
        jax.ShapeDtypeStruct((N, F), jnp.float32),
        jax.ShapeDtypeStruct((N, 1), jnp.float32),
    ],
)


def _tc_layer_body(agg_ref, h_ref, dinv_ref, w_ref, b_ref, hn_ref, gn_ref):
    dinv = dinv_ref[...]
    a = agg_ref[...] * dinv
    z = jnp.dot(a, w_ref[...]) + b_ref[...]
    hn = h_ref[...] + jnp.maximum(z, 0.0)
    hn_ref[...] = hn
    gn_ref[...] = hn * dinv


_tc_layer = pl.pallas_call(
    _tc_layer_body,
    grid=(N // RB,),
    in_specs=[
        pl.BlockSpec((RB, F), lambda i: (i, 0)),
        pl.BlockSpec((RB, F), lambda i: (i, 0)),
        pl.BlockSpec((RB, 1), lambda i: (i, 0)),
        pl.BlockSpec((F, F), lambda i: (0, 0)),
        pl.BlockSpec((1, F), lambda i: (0, 0)),
    ],
    out_specs=[
        pl.BlockSpec((RB, F), lambda i: (i, 0)),
        pl.BlockSpec((RB, F), lambda i: (i, 0)),
    ],
    out_shape=[
        jax.ShapeDtypeStruct((N, F), jnp.float32),
        jax.ShapeDtypeStruct((N, F), jnp.float32),
    ],
)


def _tc_final_body(ids_ref, h_ref, hidden_ref, l2w0_ref, l2b0_ref, l2w1_ref,
                   l2b1_ref, w1_ref, b1_ref, w2_ref, b2_ref, out_ref,
                   pooled_scr):
    i = pl.program_id(0)

    @pl.when(i == 0)
    def _():
        pooled_scr[...] = jnp.zeros_like(pooled_scr)

    onehot = (ids_ref[...] == lax.broadcasted_iota(jnp.int32, (1, B), 1)
              ).astype(jnp.float32)
    pooled_scr[...] += lax.dot_general(onehot, h_ref[...],
                                       (((0,), (0,)), ((), ())))

    @pl.when(i == pl.num_programs(0) - 1)
    def _():
        p = pooled_scr[...]
        p = jnp.maximum(jnp.dot(p, l2w0_ref[...]) + l2b0_ref[...], 0.0)
        p = jnp.maximum(jnp.dot(p, l2w1_ref[...]) + l2b1_ref[...], 0.0)
        hc = jnp.concatenate([hidden_ref[...], p], axis=1)
        hc = jnp.maximum(jnp.dot(hc, w1_ref[...]) + b1_ref[...], 0.0)
        out_ref[...] = jnp.dot(hc, w2_ref[...]) + b2_ref[...]


_tc_final = pl.pallas_call(
    _tc_final_body,
    grid=(N // RB,),
    in_specs=[
        pl.BlockSpec((RB, 1), lambda i: (i, 0)),
        pl.BlockSpec((RB, F), lambda i: (i, 0)),
        pl.BlockSpec((B, HID), lambda i: (0, 0)),
        pl.BlockSpec((F, F), lambda i: (0, 0)),
        pl.BlockSpec((1, F), lambda i: (0, 0)),
        pl.BlockSpec((F, F), lambda i: (0, 0)),
        pl.BlockSpec((1, F), lambda i: (0, 0)),
        pl.BlockSpec((HID + F, HID + F), lambda i: (0, 0)),
        pl.BlockSpec((1, HID + F), lambda i: (0, 0)),
        pl.BlockSpec((HID + F, F), lambda i: (0, 0)),
        pl.BlockSpec((1, F), lambda i: (0, 0)),
    ],
    out_specs=pl.BlockSpec((B, F), lambda i: (0, 0)),
    out_shape=jax.ShapeDtypeStruct((B, F), jnp.float32),
    scratch_shapes=[pltpu.VMEM((B, F), jnp.float32)],
)


def kernel(hidden_feats, solv_node_feats, edge_index, node_graph_ids, W_emb,
           b_emb, gcn_W, gcn_b, lin2_W, lin2_b, lin3_W1, lin3_b1, lin3_W2,
           lin3_b2):
    edge = edge_index.astype(jnp.int32)
    src = edge[0]
    dst = edge[1]
    ids = node_graph_ids.astype(jnp.int32).reshape(N, 1)
    zerosF = jnp.zeros((SLICE, F), jnp.float32)

    deg, psrc, pldst, cnt = _sc_prep(src, dst)
    h, g, dinv = _tc_init(solv_node_feats, W_emb, b_emb.reshape(1, F),
                          deg.reshape(N, 1))
    for i in range(N_GCN):
        agg = _sc_agg(g, psrc, pldst, cnt, zerosF)
        h, g = _tc_layer(agg, h, dinv, gcn_W[i], gcn_b[i].reshape(1, F))
    out = _tc_final(ids, h, hidden_feats, lin2_W[0], lin2_b[0].reshape(1, F),
                    lin2_W[1], lin2_b[1].reshape(1, F), lin3_W1,
                    lin3_b1.reshape(1, HID + F), lin3_W2,
                    lin3_b2.reshape(1, F))
    return out
